# baseline (speedup 1.0000x reference)
import jax
import jax.numpy as jnp
from jax.experimental import pallas as pl

N = 10000; E = 320000; D = 128
H1 = 4; C1 = 256; H2 = 2; C2 = 128
B = 8


def _elu_pallas(x):
    def body(x_ref, o_ref):
        v = x_ref[...]
        o_ref[...] = jnp.where(v > 0, v, jnp.exp(v) - 1.0)

    n, f = x.shape
    blk = 1000
    return pl.pallas_call(
        body,
        grid=(n // blk,),
        in_specs=[pl.BlockSpec((blk, f), lambda i: (i, 0))],
        out_specs=pl.BlockSpec((blk, f), lambda i: (i, 0)),
        out_shape=jax.ShapeDtypeStruct((n, f), x.dtype),
    )(x)


def _gat_layer(x, ei, W, a_s, a_d, bias, H, C, concat):
    n = x.shape[0]
    loops = jnp.arange(n, dtype=ei.dtype)
    src = jnp.concatenate([ei[0], loops])
    dst = jnp.concatenate([ei[1], loops])
    h = (x @ W).reshape(n, H, C)
    alpha_src = (h * a_s[None, :, :]).sum(-1)
    alpha_dst = (h * a_d[None, :, :]).sum(-1)
    e = alpha_src[src] + alpha_dst[dst]
    e = jnp.where(e > 0, e, 0.2 * e)
    m = jax.ops.segment_max(e, dst, num_segments=n)
    m = jnp.where(jnp.isfinite(m), m, 0.0)
    ex = jnp.exp(e - m[dst])
    den = jax.ops.segment_sum(ex, dst, num_segments=n)
    alpha = ex / (den[dst] + 1e-16)
    out = jax.ops.segment_sum(h[src] * alpha[:, :, None], dst, num_segments=n)
    out = out.reshape(n, H * C) if concat else out.mean(axis=1)
    return out + bias


def kernel(x, edge_index, batch_ids, W1, a_src1, a_dst1, b1, W2, a_src2, a_dst2, b2, projW, projb, gamma, beta, rW1, rb1, rW2, rb2):
    h1 = _elu_pallas(_gat_layer(x, edge_index, W1, a_src1, a_dst1, b1, H1, C1, True))
    h2 = _elu_pallas(_gat_layer(h1, edge_index, W2, a_src2, a_dst2, b2, H2, C2, False))
    counts = jax.ops.segment_sum(jnp.ones((N,), jnp.float32), batch_ids, num_segments=B)
    pooled = jax.ops.segment_sum(h2, batch_ids, num_segments=B) / jnp.maximum(counts, 1.0)[:, None]
    y = pooled @ projW + projb
    mu = y.mean(-1, keepdims=True)
    var = ((y - mu) ** 2).mean(-1, keepdims=True)
    struct = (y - mu) / jnp.sqrt(var + 1e-5) * gamma + beta
    edge_graph = batch_ids[edge_index[0]]
    ecnt = jax.ops.segment_sum(jnp.ones((E,), jnp.float32), edge_graph, num_segments=B)
    stats = jnp.stack([ecnt / (counts + 1e-6), jnp.log(counts + 1.0)], axis=1)
    risk_in = jnp.concatenate([pooled, stats], axis=-1)
    hr = jax.nn.relu(risk_in @ rW1 + rb1)
    risk = jax.nn.sigmoid(hr @ rW2 + rb2).squeeze(-1)
    return struct, h2, risk


# speedup vs baseline: 7.8874x; 7.8874x over previous
import functools
import jax
import jax.numpy as jnp
from jax import lax
from jax.experimental import pallas as pl
from jax.experimental.pallas import tpu as pltpu
from jax.experimental.pallas import tpu_sc as plsc

N = 10000; E = 320000; D = 128
H1 = 4; C1 = 256; H2 = 2; C2 = 128
B = 8

NC = 2
NS = 16
NW = NC * NS
R = 64
NCH = 160
CPW = NCH // NW
NP = NCH * R
ET = E + N
BLKE = 1024
EPAD = ET + BLKE + 8


def _gat_edge_sc(H, C, concat):
    HC = H * C

    def body(src_hbm, dst_hbm, offs_hbm, h_hbm, asrc_hbm, adst_hbm, bias_hbm,
             out_hbm,
             asrc_v, adstc_v, offs_v, srcb_v, dstb_v, idx16_v,
             rows_v, acc_v, den_v, bias_v, outc_v, sem):
        wid = lax.axis_index("s") * NC + lax.axis_index("c")
        pltpu.sync_copy(asrc_hbm, asrc_v)
        pltpu.sync_copy(offs_hbm, offs_v)
        pltpu.sync_copy(bias_hbm, bias_v)
        iota = lax.iota(jnp.int32, 16)

        def chunk_body(cc, _):
            c = wid * CPW + cc
            rowbase = c * R
            pltpu.sync_copy(adst_hbm.at[pl.ds(rowbase * H, R * H)], adstc_v)

            def zero_acc(i, _):
                acc_v[pl.ds(i * 16, 16)] = jnp.zeros((16,), jnp.float32)
                return 0
            lax.fori_loop(0, R * HC // 16, zero_acc, 0)

            def zero_den(i, _):
                den_v[pl.ds(i * 16, 16)] = jnp.zeros((16,), jnp.float32)
                return 0
            lax.fori_loop(0, R, zero_den, 0)

            ev = plsc.load_gather(offs_v, [jnp.full((16,), c, jnp.int32)])
            ev1 = plsc.load_gather(
                offs_v, [jnp.full((16,), c + 1, jnp.int32)])
            e_start = jnp.max(ev)
            e_end = jnp.max(ev1)
            a_start = (e_start // 8) * 8
            nblocks = (e_end - a_start + BLKE - 1) // BLKE

            def block_body(b, _):
                bbase = a_start + b * BLKE
                pltpu.sync_copy(src_hbm.at[pl.ds(bbase, BLKE)], srcb_v)
                pltpu.sync_copy(dst_hbm.at[pl.ds(bbase, BLKE)], dstb_v)
                rem = e_end - bbase
                ng = (jnp.minimum(rem, BLKE) + 15) // 16

                def group_body(g, _):
                    gb = g * 16
                    src16 = srcb_v[pl.ds(gb, 16)]
                    dst16 = dstb_v[pl.ds(gb, 16)]
                    gidx = bbase + gb + iota
                    valid = (gidx >= e_start) & (gidx < e_end)
                    dloc = jnp.clip(dst16 - rowbase, 0, R - 1)
                    srcc = jnp.where(valid, src16, 0)
                    idx16_v[pl.ds(0, 16)] = srcc
                    exs_h = []
                    for h in range(H):
                        av = plsc.load_gather(asrc_v, [srcc * H + h])
                        bv = plsc.load_gather(adstc_v, [dloc * H + h])
                        e = av + bv
                        e = jnp.where(e > 0, e, 0.2 * e)
                        exs_h.append(jnp.where(valid, jnp.exp(e), 0.0))
                    cp = pltpu.async_copy(h_hbm.at[idx16_v], rows_v, sem)
                    cp.wait()
                    for i in range(16):
                        dl = dloc[i]
                        dbase = dl * 16
                        dv = den_v[pl.ds(dbase, 16)]
                        exrow = jnp.zeros((16,), jnp.float32)
                        for h in range(H):
                            exs = exs_h[h][i]
                            exrow = jnp.where(iota == h, exs, exrow)
                            sc = jnp.full((16,), exs, jnp.float32)
                            abase = dl * HC + h * C
                            for j in range(C // 16):
                                col = h * C + j * 16
                                v = rows_v[i, pl.ds(col, 16)]
                                a = acc_v[pl.ds(abase + j * 16, 16)]
                                acc_v[pl.ds(abase + j * 16, 16)] = a + v * sc
                        den_v[pl.ds(dbase, 16)] = dv + exrow
                    return 0
                lax.fori_loop(0, ng, group_body, 0)
                return 0
            lax.fori_loop(0, nblocks, block_body, 0)

            def fin_body(r, _):
                dv = den_v[pl.ds(r * 16, 16)]
                invv = jnp.ones((16,), jnp.float32) / (dv + 1e-16)
                if concat:
                    for h in range(H):
                        inv = jnp.full((16,), invv[h], jnp.float32)
                        for j in range(C // 16):
                            idx = r * HC + h * C + j * 16
                            v = acc_v[pl.ds(idx, 16)] * inv
                            v = v + bias_v[pl.ds(h * C + j * 16, 16)]
                            v = jnp.where(v > 0, v, jnp.exp(v) - 1.0)
                            acc_v[pl.ds(idx, 16)] = v
                else:
                    inv0 = jnp.full((16,), invv[0], jnp.float32) * 0.5
                    inv1 = jnp.full((16,), invv[1], jnp.float32) * 0.5
                    for j in range(C // 16):
                        v0 = acc_v[pl.ds(r * HC + j * 16, 16)] * inv0
                        v1 = acc_v[pl.ds(r * HC + C + j * 16, 16)] * inv1
                        v = v0 + v1 + bias_v[pl.ds(j * 16, 16)]
                        v = jnp.where(v > 0, v, jnp.exp(v) - 1.0)
                        outc_v[pl.ds(r * C + j * 16, 16)] = v
                return 0
            lax.fori_loop(0, R, fin_body, 0)

            W = HC if concat else C
            srcv = acc_v if concat else outc_v
            pltpu.sync_copy(srcv, out_hbm.at[pl.ds(rowbase * W, R * W)])
            return 0
        lax.fori_loop(0, CPW, chunk_body, 0)

    W = HC if concat else C
    mesh = plsc.VectorSubcoreMesh(core_axis_name="c", subcore_axis_name="s")
    kern = pl.kernel(
        body,
        out_type=jax.ShapeDtypeStruct((NP * W,), jnp.float32),
        mesh=mesh,
        compiler_params=pltpu.CompilerParams(needs_layout_passes=False),
        scratch_types=[
            pltpu.VMEM((NP * H,), jnp.float32),
            pltpu.VMEM((R * H,), jnp.float32),
            pltpu.VMEM((NCH + 8,), jnp.int32),
            pltpu.VMEM((BLKE,), jnp.int32),
            pltpu.VMEM((BLKE,), jnp.int32),
            pltpu.VMEM((16,), jnp.int32),
            pltpu.VMEM((16, HC), jnp.float32),
            pltpu.VMEM((R * HC,), jnp.float32),
            pltpu.VMEM((R * 16,), jnp.float32),
            pltpu.VMEM((HC if concat else C,), jnp.float32),
            pltpu.VMEM((16 if concat else R * C,), jnp.float32),
            pltpu.SemaphoreType.DMA,
        ],
    )

    def run(src, dst, offs, h, asrc, adst, bias):
        return kern(src, dst, offs, h, asrc, adst, bias)
    return run


_sc_l1 = _gat_edge_sc(H1, C1, True)
_sc_l2 = _gat_edge_sc(H2, C2, False)


def _matmul_tc(x, w, bm):
    M, K = x.shape
    NO = w.shape[1]

    def body(x_ref, w_ref, o_ref):
        o_ref[...] = jnp.dot(x_ref[...], w_ref[...],
                             preferred_element_type=jnp.float32)

    return pl.pallas_call(
        body,
        grid=(M // bm,),
        in_specs=[pl.BlockSpec((bm, K), lambda i: (i, 0)),
                  pl.BlockSpec((K, NO), lambda i: (0, 0))],
        out_specs=pl.BlockSpec((bm, NO), lambda i: (i, 0)),
        out_shape=jax.ShapeDtypeStruct((M, NO), jnp.float32),
    )(x, w)


def _fold_attn(Wm, a_s, a_d, H, C):
    HC = H * C
    A = jnp.zeros((HC, 128), jnp.float32)
    Bm = jnp.zeros((HC, 128), jnp.float32)
    for h in range(H):
        A = A.at[h * C:(h + 1) * C, h].set(a_s[h])
        Bm = Bm.at[h * C:(h + 1) * C, h].set(a_d[h])
    return jnp.concatenate([Wm, Wm @ A, Wm @ Bm], axis=1)


def _gat_layer(h_prev, src_s, dst_s, offs, Wcat, bias, H, C, concat):
    HC = H * C
    y = _matmul_tc(h_prev, Wcat, 512)
    h = y[:, :HC]
    asrc = y[:, HC:HC + H].reshape(-1)
    adst = y[:, HC + 128:HC + 128 + H].reshape(-1)
    out = _gat_edge_sc_call(H, C, concat, src_s, dst_s, offs, h, asrc, adst,
                            bias)
    W = HC if concat else C
    return out.reshape(NP, W)


def _gat_edge_sc_call(H, C, concat, src_s, dst_s, offs, h, asrc, adst, bias):
    if concat:
        return _sc_l1(src_s, dst_s, offs, h, asrc, adst, bias)
    return _sc_l2(src_s, dst_s, offs, h, asrc, adst, bias)


def kernel(x, edge_index, batch_ids, W1, a_src1, a_dst1, b1, W2, a_src2,
           a_dst2, b2, projW, projb, gamma, beta, rW1, rb1, rW2, rb2):
    loops = jnp.arange(N, dtype=edge_index.dtype)
    src_all = jnp.concatenate([edge_index[0], loops]).astype(jnp.int32)
    dst_all = jnp.concatenate([edge_index[1], loops]).astype(jnp.int32)
    order = jnp.argsort(dst_all)
    src_s = src_all[order]
    dst_s = dst_all[order]
    offs = jnp.searchsorted(
        dst_s, jnp.arange(NCH + 1, dtype=jnp.int32) * R).astype(jnp.int32)
    offs = jnp.concatenate([offs, jnp.full((7,), ET, jnp.int32)])
    src_p = jnp.concatenate([src_s, jnp.zeros((EPAD - ET,), jnp.int32)])
    dst_p = jnp.concatenate([dst_s, jnp.zeros((EPAD - ET,), jnp.int32)])

    xp = jnp.concatenate([x, jnp.zeros((NP - N, D), jnp.float32)], axis=0)
    Wcat1 = _fold_attn(W1, a_src1, a_dst1, H1, C1)
    Wcat2 = _fold_attn(W2, a_src2, a_dst2, H2, C2)

    h1 = _gat_layer(xp, src_p, dst_p, offs, Wcat1, b1, H1, C1, True)
    h2p = _gat_layer(h1, src_p, dst_p, offs, Wcat2, b2, H2, C2, False)
    h2 = h2p[:N]

    counts = jax.ops.segment_sum(jnp.ones((N,), jnp.float32), batch_ids,
                                 num_segments=B)
    pooled = jax.ops.segment_sum(h2, batch_ids, num_segments=B) \
        / jnp.maximum(counts, 1.0)[:, None]
    y = pooled @ projW + projb
    mu = y.mean(-1, keepdims=True)
    var = ((y - mu) ** 2).mean(-1, keepdims=True)
    struct = (y - mu) / jnp.sqrt(var + 1e-5) * gamma + beta
    edge_graph = batch_ids[edge_index[0]]
    ecnt = jax.ops.segment_sum(jnp.ones((E,), jnp.float32), edge_graph,
                               num_segments=B)
    stats = jnp.stack([ecnt / (counts + 1e-6), jnp.log(counts + 1.0)], axis=1)
    risk_in = jnp.concatenate([pooled, stats], axis=-1)
    hr = jax.nn.relu(risk_in @ rW1 + rb1)
    risk = jax.nn.sigmoid(hr @ rW2 + rb2).squeeze(-1)
    return struct, h2, risk
